# baseline (speedup 1.0000x reference)
import jax
import jax.numpy as jnp
from jax import lax
from jax.experimental import pallas as pl
from jax.experimental.pallas import tpu as pltpu
from jax.experimental.pallas import tpu_sc as plsc

B, C, H, W = 2, 96, 512, 512
G = B * C
NTC = 168
NSC = G - NTC
HB = 128
NH = H // HB
WP = W + 8

L = 16
NC, NS = 2, 16
NWORK = NC * NS
HBS = 64
NHC = H // HBS
CPW = NSC * NHC // NWORK
XW = 544
NWV = W // L
NXV = XW // L


def _coef_body(w_ref, psi_ref, coef_ref):
    coef_ref[...] = jax.lax.dot(
        w_ref[...], psi_ref[...], preferred_element_type=jnp.float32
    )


def _tc_body(coef_ref, bias_ref, x_ref, o_ref, xp_ref):
    g = pl.program_id(0)
    h = pl.program_id(1)
    c = g % C

    xp_ref[...] = jnp.zeros((HB + 16, WP), jnp.float32)

    @pl.when(h == 0)
    def _():
        xp_ref[8:HB + 16, 2:W + 2] = x_ref[0, 0:HB + 8, :]

    @pl.when(h == NH - 1)
    def _():
        xp_ref[0:HB + 8, 2:W + 2] = x_ref[0, H - HB - 8:H, :]

    @pl.when(jnp.logical_and(h > 0, h < NH - 1))
    def _():
        start = pl.multiple_of(h * HB - 8, 8)
        xp_ref[0:HB + 16, 2:W + 2] = x_ref[0, pl.ds(start, HB + 16), :]

    rows = [xp_ref[6 + dy:6 + dy + HB, :] for dy in range(5)]
    s04 = rows[0] + rows[4]
    s13 = rows[1] + rows[3]
    s2 = rows[2]
    tmps = []
    for dx in range(3):
        tmp = (coef_ref[c, dx] * s04
               + coef_ref[c, 5 + dx] * s13
               + coef_ref[c, 10 + dx] * s2)
        tmps.append(tmp)

    acc = jnp.full((HB, W), bias_ref[c], jnp.float32)
    acc = acc + tmps[0][:, 0:W] + tmps[1][:, 1:1 + W] + tmps[2][:, 2:2 + W]
    acc = acc + tmps[1][:, 3:3 + W] + tmps[0][:, 4:4 + W]
    o_ref[0] = acc


def _sc_body(x_hbm, coefb_hbm, biasb_hbm, out_hbm, xw, ow, tw, cw):
    wid = lax.axis_index("s") * NC + lax.axis_index("c")

    def chunk_body(j, _):
        cid = wid * CPW + j
        gi = NTC + cid // NHC
        h = cid % NHC
        c = gi % C

        pltpu.sync_copy(coefb_hbm.at[c], cw.at[0:25])
        pltpu.sync_copy(biasb_hbm.at[c], cw.at[25])

        zv = jnp.zeros((L,), jnp.float32)
        for r in range(HBS + 4):
            xw[r, pl.ds(0, L)] = zv
            xw[r, pl.ds(XW - L, L)] = zv

        @pl.when(h == 0)
        def _():
            for r in range(2):
                for v in range(NXV):
                    xw[r, pl.ds(v * L, L)] = zv
            pltpu.sync_copy(x_hbm.at[gi, pl.ds(0, HBS + 2), :],
                            xw.at[pl.ds(2, HBS + 2), pl.ds(16, W)])

        @pl.when(h == NHC - 1)
        def _():
            for r in range(HBS + 2, HBS + 4):
                for v in range(NXV):
                    xw[r, pl.ds(v * L, L)] = zv
            pltpu.sync_copy(x_hbm.at[gi, pl.ds(H - HBS - 2, HBS + 2), :],
                            xw.at[pl.ds(0, HBS + 2), pl.ds(16, W)])

        @pl.when(jnp.logical_and(h > 0, h < NHC - 1))
        def _():
            pltpu.sync_copy(x_hbm.at[gi, pl.ds(h * HBS - 2, HBS + 4), :],
                            xw.at[:, pl.ds(16, W)])

        cq = [[cw[5 * dy + dx, :] for dx in range(3)] for dy in range(3)]
        biasv = cw[25, :]

        def row_body(i, _):
            for v in range(NXV):
                r0 = xw[i + 0, pl.ds(v * L, L)]
                r1 = xw[i + 1, pl.ds(v * L, L)]
                r2 = xw[i + 2, pl.ds(v * L, L)]
                r3 = xw[i + 3, pl.ds(v * L, L)]
                r4 = xw[i + 4, pl.ds(v * L, L)]
                s04 = r0 + r4
                s13 = r1 + r3
                for dx in range(3):
                    tw[dx, pl.ds(v * L, L)] = (
                        cq[0][dx] * s04 + cq[1][dx] * s13 + cq[2][dx] * r2
                    )
            for wv in range(NWV):
                base = 16 + wv * L
                acc = biasv + tw[0, pl.ds(base - 2, L)]
                acc = acc + tw[1, pl.ds(base - 1, L)]
                acc = acc + tw[2, pl.ds(base, L)]
                acc = acc + tw[1, pl.ds(base + 1, L)]
                acc = acc + tw[0, pl.ds(base + 2, L)]
                ow[i, pl.ds(wv * L, L)] = acc
            return 0

        lax.fori_loop(0, HBS, row_body, 0)

        pltpu.sync_copy(ow, out_hbm.at[gi - NTC, pl.ds(h * HBS, HBS), :])
        return 0

    lax.fori_loop(0, CPW, chunk_body, 0)


def kernel(x, weight, bias, psi_loc):
    w2 = weight.reshape(C, -1)[:, -3:]
    psi25 = psi_loc[:, 1:6, 1:6].reshape(3, 25)

    coef = pl.pallas_call(
        _coef_body,
        out_shape=jax.ShapeDtypeStruct((C, 25), jnp.float32),
    )(w2, psi25)
    coefb = jnp.broadcast_to(coef[:, :, None], (C, 25, L))
    biasb = jnp.broadcast_to(bias[:, None], (C, L))

    x2 = x.reshape(G, H, W)

    out_sc = pl.kernel(
        _sc_body,
        out_type=jax.ShapeDtypeStruct((NSC, H, W), jnp.float32),
        mesh=plsc.VectorSubcoreMesh(core_axis_name="c", subcore_axis_name="s"),
        scratch_types=[
            pltpu.VMEM((HBS + 4, XW), jnp.float32),
            pltpu.VMEM((HBS, W), jnp.float32),
            pltpu.VMEM((3, XW), jnp.float32),
            pltpu.VMEM((26, L), jnp.float32),
        ],
        compiler_params=pltpu.CompilerParams(use_tc_tiling_on_sc=False),
    )(x2, coefb, biasb)

    out_tc = pl.pallas_call(
        _tc_body,
        grid=(NTC, NH),
        in_specs=[
            pl.BlockSpec(memory_space=pltpu.SMEM),
            pl.BlockSpec(memory_space=pltpu.SMEM),
            pl.BlockSpec((1, H, W), lambda g, h: (g, 0, 0)),
        ],
        out_specs=pl.BlockSpec((1, HB, W), lambda g, h: (g, h, 0)),
        out_shape=jax.ShapeDtypeStruct((NTC, H, W), jnp.float32),
        scratch_shapes=[pltpu.VMEM((HB + 16, WP), jnp.float32)],
    )(coef, bias, x2)

    return jnp.concatenate([out_tc, out_sc], axis=0).reshape(B, C, H, W)


# speedup vs baseline: 1.0008x; 1.0008x over previous
import jax
import jax.numpy as jnp
from jax import lax
from jax.experimental import compute_on
from jax.experimental import pallas as pl
from jax.experimental.pallas import tpu as pltpu
from jax.experimental.pallas import tpu_sc as plsc

B, C, H, W = 2, 96, 512, 512
G = B * C
NTC = 168
NSC = G - NTC
HB = 128
NH = H // HB
WP = W + 8

L = 16
NC, NS = 2, 16
NWORK = NC * NS
HBS = 64
NHC = H // HBS
CPW = NSC * NHC // NWORK
XW = 544
NWV = W // L
NXV = XW // L


def _coef_body(w_ref, psi_ref, coef_ref):
    coef_ref[...] = jax.lax.dot(
        w_ref[...], psi_ref[...], preferred_element_type=jnp.float32
    )


def _tc_body(coef_ref, bias_ref, x_ref, o_ref, xp_ref):
    g = pl.program_id(0)
    h = pl.program_id(1)
    c = g % C

    xp_ref[...] = jnp.zeros((HB + 16, WP), jnp.float32)

    @pl.when(h == 0)
    def _():
        xp_ref[8:HB + 16, 2:W + 2] = x_ref[0, 0:HB + 8, :]

    @pl.when(h == NH - 1)
    def _():
        xp_ref[0:HB + 8, 2:W + 2] = x_ref[0, H - HB - 8:H, :]

    @pl.when(jnp.logical_and(h > 0, h < NH - 1))
    def _():
        start = pl.multiple_of(h * HB - 8, 8)
        xp_ref[0:HB + 16, 2:W + 2] = x_ref[0, pl.ds(start, HB + 16), :]

    rows = [xp_ref[6 + dy:6 + dy + HB, :] for dy in range(5)]
    s04 = rows[0] + rows[4]
    s13 = rows[1] + rows[3]
    s2 = rows[2]
    tmps = []
    for dx in range(3):
        tmp = (coef_ref[c, dx] * s04
               + coef_ref[c, 5 + dx] * s13
               + coef_ref[c, 10 + dx] * s2)
        tmps.append(tmp)

    acc = jnp.full((HB, W), bias_ref[c], jnp.float32)
    acc = acc + tmps[0][:, 0:W] + tmps[1][:, 1:1 + W] + tmps[2][:, 2:2 + W]
    acc = acc + tmps[1][:, 3:3 + W] + tmps[0][:, 4:4 + W]
    o_ref[0] = acc


def _sc_body(x_hbm, coefb_hbm, biasb_hbm, out_hbm, xw, ow, tw, cw):
    wid = lax.axis_index("s") * NC + lax.axis_index("c")

    def chunk_body(j, _):
        cid = wid * CPW + j
        gi = NTC + cid // NHC
        h = cid % NHC
        c = gi % C

        pltpu.sync_copy(coefb_hbm.at[c], cw.at[0:25])
        pltpu.sync_copy(biasb_hbm.at[c], cw.at[25])

        zv = jnp.zeros((L,), jnp.float32)
        for r in range(HBS + 4):
            xw[r, pl.ds(0, L)] = zv
            xw[r, pl.ds(XW - L, L)] = zv

        @pl.when(h == 0)
        def _():
            for r in range(2):
                for v in range(NXV):
                    xw[r, pl.ds(v * L, L)] = zv
            pltpu.sync_copy(x_hbm.at[gi, pl.ds(0, HBS + 2), :],
                            xw.at[pl.ds(2, HBS + 2), pl.ds(16, W)])

        @pl.when(h == NHC - 1)
        def _():
            for r in range(HBS + 2, HBS + 4):
                for v in range(NXV):
                    xw[r, pl.ds(v * L, L)] = zv
            pltpu.sync_copy(x_hbm.at[gi, pl.ds(H - HBS - 2, HBS + 2), :],
                            xw.at[pl.ds(0, HBS + 2), pl.ds(16, W)])

        @pl.when(jnp.logical_and(h > 0, h < NHC - 1))
        def _():
            pltpu.sync_copy(x_hbm.at[gi, pl.ds(h * HBS - 2, HBS + 4), :],
                            xw.at[:, pl.ds(16, W)])

        cq = [[cw[5 * dy + dx, :] for dx in range(3)] for dy in range(3)]
        biasv = cw[25, :]

        def row_body(i, _):
            for v in range(NXV):
                r0 = xw[i + 0, pl.ds(v * L, L)]
                r1 = xw[i + 1, pl.ds(v * L, L)]
                r2 = xw[i + 2, pl.ds(v * L, L)]
                r3 = xw[i + 3, pl.ds(v * L, L)]
                r4 = xw[i + 4, pl.ds(v * L, L)]
                s04 = r0 + r4
                s13 = r1 + r3
                for dx in range(3):
                    tw[dx, pl.ds(v * L, L)] = (
                        cq[0][dx] * s04 + cq[1][dx] * s13 + cq[2][dx] * r2
                    )
            for wv in range(NWV):
                base = 16 + wv * L
                acc = biasv + tw[0, pl.ds(base - 2, L)]
                acc = acc + tw[1, pl.ds(base - 1, L)]
                acc = acc + tw[2, pl.ds(base, L)]
                acc = acc + tw[1, pl.ds(base + 1, L)]
                acc = acc + tw[0, pl.ds(base + 2, L)]
                ow[i, pl.ds(wv * L, L)] = acc
            return 0

        lax.fori_loop(0, HBS, row_body, 0)

        pltpu.sync_copy(ow, out_hbm.at[gi - NTC, pl.ds(h * HBS, HBS), :])
        return 0

    lax.fori_loop(0, CPW, chunk_body, 0)


def kernel(x, weight, bias, psi_loc):
    w2 = weight.reshape(C, -1)[:, -3:]
    psi25 = psi_loc[:, 1:6, 1:6].reshape(3, 25)

    coef = pl.pallas_call(
        _coef_body,
        out_shape=jax.ShapeDtypeStruct((C, 25), jnp.float32),
    )(w2, psi25)
    coefb = jnp.broadcast_to(coef[:, :, None], (C, 25, L))
    biasb = jnp.broadcast_to(bias[:, None], (C, L))

    x2 = x.reshape(G, H, W)

    with compute_on.compute_on("tpu_sparsecore"):
        out_sc = pl.kernel(
            _sc_body,
            out_type=jax.ShapeDtypeStruct((NSC, H, W), jnp.float32),
            mesh=plsc.VectorSubcoreMesh(core_axis_name="c", subcore_axis_name="s"),
            scratch_types=[
                pltpu.VMEM((HBS + 4, XW), jnp.float32),
                pltpu.VMEM((HBS, W), jnp.float32),
                pltpu.VMEM((3, XW), jnp.float32),
                pltpu.VMEM((26, L), jnp.float32),
            ],
            compiler_params=pltpu.CompilerParams(use_tc_tiling_on_sc=False),
        )(x2, coefb, biasb)

    out_tc = pl.pallas_call(
        _tc_body,
        grid=(NTC, NH),
        in_specs=[
            pl.BlockSpec(memory_space=pltpu.SMEM),
            pl.BlockSpec(memory_space=pltpu.SMEM),
            pl.BlockSpec((1, H, W), lambda g, h: (g, 0, 0)),
        ],
        out_specs=pl.BlockSpec((1, HB, W), lambda g, h: (g, h, 0)),
        out_shape=jax.ShapeDtypeStruct((NTC, H, W), jnp.float32),
        scratch_shapes=[pltpu.VMEM((HB + 16, WP), jnp.float32)],
    )(coef, bias, x2)

    return jnp.concatenate([out_tc, out_sc], axis=0).reshape(B, C, H, W)


# speedup vs baseline: 1.2777x; 1.2767x over previous
import jax
import jax.numpy as jnp
from jax.experimental import pallas as pl
from jax.experimental.pallas import tpu as pltpu

B, C, H, W = 2, 96, 512, 512
HB = 128
NH = H // HB
WP = W + 8


def _coef_body(w_ref, psi_ref, coef_ref):
    coef_ref[...] = jax.lax.dot(
        w_ref[...], psi_ref[...], preferred_element_type=jnp.float32
    )


def _conv_body(coef_ref, bias_ref, x_ref, o_ref, xp_ref):
    c = pl.program_id(1)
    h = pl.program_id(2)

    xp_ref[:, 0:2] = jnp.zeros((HB + 16, 2), jnp.float32)
    xp_ref[:, W + 2:WP] = jnp.zeros((HB + 16, WP - W - 2), jnp.float32)

    @pl.when(h == 0)
    def _():
        xp_ref[0:8, :] = jnp.zeros((8, WP), jnp.float32)
        xp_ref[8:HB + 16, 2:W + 2] = x_ref[0, 0, 0:HB + 8, :]

    @pl.when(h == NH - 1)
    def _():
        xp_ref[HB + 8:HB + 16, :] = jnp.zeros((8, WP), jnp.float32)
        xp_ref[0:HB + 8, 2:W + 2] = x_ref[0, 0, H - HB - 8:H, :]

    @pl.when(jnp.logical_and(h > 0, h < NH - 1))
    def _():
        start = pl.multiple_of(h * HB - 8, 8)
        xp_ref[0:HB + 16, 2:W + 2] = x_ref[0, 0, pl.ds(start, HB + 16), :]

    rows = [xp_ref[6 + dy:6 + dy + HB, :] for dy in range(5)]
    s04 = rows[0] + rows[4]
    s13 = rows[1] + rows[3]
    s2 = rows[2]
    tmps = []
    for dx in range(3):
        tmp = (coef_ref[c, dx] * s04
               + coef_ref[c, 5 + dx] * s13
               + coef_ref[c, 10 + dx] * s2)
        tmps.append(tmp)

    acc = jnp.full((HB, W), bias_ref[c], jnp.float32)
    acc = acc + tmps[0][:, 0:W] + tmps[1][:, 1:1 + W] + tmps[2][:, 2:2 + W]
    acc = acc + tmps[1][:, 3:3 + W] + tmps[0][:, 4:4 + W]
    o_ref[0, 0] = acc


def kernel(x, weight, bias, psi_loc):
    w2 = weight.reshape(C, -1)[:, -3:]
    psi25 = psi_loc[:, 1:6, 1:6].reshape(3, 25)

    coef = pl.pallas_call(
        _coef_body,
        out_shape=jax.ShapeDtypeStruct((C, 25), jnp.float32),
    )(w2, psi25)

    out = pl.pallas_call(
        _conv_body,
        grid=(B, C, NH),
        in_specs=[
            pl.BlockSpec(memory_space=pltpu.SMEM),
            pl.BlockSpec(memory_space=pltpu.SMEM),
            pl.BlockSpec((1, 1, H, W), lambda b, c, h: (b, c, 0, 0)),
        ],
        out_specs=pl.BlockSpec((1, 1, HB, W), lambda b, c, h: (b, c, h, 0)),
        out_shape=jax.ShapeDtypeStruct((B, C, H, W), jnp.float32),
        scratch_shapes=[pltpu.VMEM((HB + 16, WP), jnp.float32)],
    )(coef, bias, x)
    return out
